# baseline (speedup 1.0000x reference)
import functools

import jax
import jax.numpy as jnp
from jax import lax
from jax.experimental import pallas as pl
from jax.experimental.pallas import tpu as pltpu
from jax.experimental.pallas import tpu_sc as plsc

_INFO = plsc.get_sparse_core_info()
_NW = _INFO.num_cores * _INFO.num_subcores
_NBUF = 2


@functools.lru_cache(maxsize=None)
def _make_gather(total: int, dim: int):
    assert total % _NW == 0
    b_per_w = total // _NW
    chunk = 1600
    while b_per_w % chunk:
        chunk //= 2
    n_chunks = b_per_w // chunk
    mesh = plsc.VectorSubcoreMesh(core_axis_name="c", subcore_axis_name="s")

    scratch = (
        [pltpu.VMEM((chunk,), jnp.int32) for _ in range(_NBUF)]
        + [pltpu.VMEM((chunk, dim), jnp.float32) for _ in range(_NBUF)]
        + [pltpu.SemaphoreType.DMA for _ in range(3 * _NBUF)]
    )

    @functools.partial(
        pl.kernel,
        mesh=mesh,
        compiler_params=pltpu.CompilerParams(use_tc_tiling_on_sc=False),
        out_type=jax.ShapeDtypeStruct((total, dim), jnp.float32),
        scratch_types=scratch,
    )
    def gather(idx_hbm, table_hbm, out_hbm, *refs):
        idx_v = refs[:_NBUF]
        rows_v = refs[_NBUF:2 * _NBUF]
        sem_i = refs[2 * _NBUF:3 * _NBUF]
        sem_g = refs[3 * _NBUF:4 * _NBUF]
        sem_s = refs[4 * _NBUF:5 * _NBUF]

        wid = lax.axis_index("s") * _INFO.num_cores + lax.axis_index("c")
        base = wid * b_per_w

        def off(i):
            return pl.multiple_of(base + i * chunk, 8)

        idx_dma = [None] * _NBUF
        gat_dma = [None] * _NBUF
        st_dma = [None] * _NBUF

        for b in range(min(_NBUF, n_chunks)):
            idx_dma[b] = pltpu.async_copy(
                idx_hbm.at[pl.ds(off(b), chunk)], idx_v[b], sem_i[b])

        for i in range(n_chunks):
            b = i % _NBUF
            idx_dma[b].wait()
            if i >= _NBUF:
                st_dma[b].wait()
            gat_dma[b] = pltpu.async_copy(
                table_hbm.at[idx_v[b]], rows_v[b], sem_g[b])
            gat_dma[b].wait()
            st_dma[b] = pltpu.async_copy(
                rows_v[b], out_hbm.at[pl.ds(off(i), chunk)], sem_s[b])
            nxt = i + _NBUF
            if nxt < n_chunks:
                idx_dma[b] = pltpu.async_copy(
                    idx_hbm.at[pl.ds(off(nxt), chunk)], idx_v[b], sem_i[b])

        for i in range(max(0, n_chunks - _NBUF), n_chunks):
            st_dma[i % _NBUF].wait()

    return gather


def kernel(x, table):
    b, h = x.shape
    dim = table.shape[1]
    idx = x.reshape(-1).astype(jnp.int32)
    out = _make_gather(b * h, dim)(idx, table)
    return out.reshape(b, h, dim)


# speedup vs baseline: 1.2999x; 1.2999x over previous
import functools

import jax
import jax.numpy as jnp
from jax import lax
from jax.experimental import pallas as pl
from jax.experimental.pallas import tpu as pltpu
from jax.experimental.pallas import tpu_sc as plsc

_INFO = plsc.get_sparse_core_info()
_NW = _INFO.num_cores * _INFO.num_subcores
_L = _INFO.num_lanes
_BB = 256


@functools.lru_cache(maxsize=None)
def _make_gather(batch: int, hist: int, dim: int):
    n_blocks = (batch // _BB) * hist
    assert batch % _BB == 0 and n_blocks % (2 * _NW) == 0
    blk_per_w = n_blocks // _NW
    n_pairs = blk_per_w // 2
    bb_per_h = batch // _BB
    mesh = plsc.VectorSubcoreMesh(core_axis_name="c", subcore_axis_name="s")

    scratch = (
        [pltpu.VMEM((_BB,), jnp.int32) for _ in range(2)]
        + [pltpu.VMEM((_BB,), jnp.int32) for _ in range(2)]
        + [pltpu.VMEM((_BB, dim), jnp.float32) for _ in range(2)]
        + [pltpu.VMEM((dim, _BB), jnp.float32) for _ in range(2)]
        + [pltpu.SemaphoreType.DMA for _ in range(6)]
    )

    @functools.partial(
        pl.kernel,
        mesh=mesh,
        compiler_params=pltpu.CompilerParams(
            use_tc_tiling_on_sc=False, needs_layout_passes=False),
        out_type=jax.ShapeDtypeStruct((hist, dim, batch), jnp.float32),
        scratch_types=scratch,
    )
    def gather(xt_hbm, table_hbm, out_hbm, *refs):
        idx_v = refs[0:2]
        line_v = refs[2:4]
        rows_v = refs[4:6]
        outb_v = refs[6:8]
        sem_i = refs[8:10]
        sem_g = refs[10:12]
        sem_s = refs[12:14]

        wid = lax.axis_index("s") * _INFO.num_cores + lax.axis_index("c")
        t0 = wid * blk_per_w
        t_last = t0 + blk_per_w - 1

        def hb(t):
            return t // bb_per_h, (t % bb_per_h) * _BB

        def start_idx(t, b):
            h, b0 = hb(t)
            return pltpu.async_copy(
                xt_hbm.at[h, pl.ds(b0, _BB)], idx_v[b], sem_i[b])

        def compute_lines(b):
            for u in range(_BB // _L):
                sl = pl.ds(u * _L, _L)
                line_v[b][sl] = idx_v[b][sl]

        def start_gather(b):
            return pltpu.async_copy(
                table_hbm.at[line_v[b]], rows_v[b], sem_g[b])

        def wait_gather(b):
            pltpu.make_async_copy(
                table_hbm.at[line_v[b]], rows_v[b], sem_g[b]).wait()

        def extract(b):
            for u in range(_BB // _L):
                sl = pl.ds(u * _L, _L)
                row_ids = lax.iota(jnp.int32, _L) + jnp.int32(u * _L)
                for d in range(dim):
                    outb_v[b][d, sl] = plsc.load_gather(
                        rows_v[b], [row_ids, jnp.full((_L,), d, jnp.int32)])

        def start_store(t, b):
            h, b0 = hb(t)
            return pltpu.async_copy(
                outb_v[b], out_hbm.at[h, :, pl.ds(b0, _BB)], sem_s[b])

        def wait_store(t, b):
            h, b0 = hb(t)
            pltpu.make_async_copy(
                outb_v[b], out_hbm.at[h, :, pl.ds(b0, _BB)], sem_s[b]).wait()

        start_idx(t0, 0).wait()
        compute_lines(0)
        start_gather(0)
        start_idx(t0 + 1, 1).wait()
        compute_lines(1)
        start_gather(1)
        i2 = start_idx(t0 + 2, 0)
        i3 = start_idx(t0 + 3, 1)
        wait_gather(0)
        extract(0)
        start_store(t0, 0)
        i2.wait()
        compute_lines(0)
        start_gather(0)
        wait_gather(1)
        extract(1)
        start_store(t0 + 1, 1)
        i3.wait()
        compute_lines(1)
        start_gather(1)

        def body(g, carry):
            i0 = t0 + 2 * g
            not_last = jnp.int32(g) < jnp.int32(n_pairs - 1)
            d0 = start_idx(jnp.minimum(i0 + 2, t_last), 0)
            wait_gather(0)
            wait_store(i0 - 2, 0)
            extract(0)
            start_store(i0, 0)
            d0.wait()
            compute_lines(0)

            @pl.when(not_last)
            def _():
                start_gather(0)

            d1 = start_idx(jnp.minimum(i0 + 3, t_last), 1)
            wait_gather(1)
            wait_store(i0 - 1, 1)
            extract(1)
            start_store(i0 + 1, 1)
            d1.wait()
            compute_lines(1)

            @pl.when(not_last)
            def _():
                start_gather(1)

            return carry

        lax.fori_loop(1, n_pairs, body, 0)

        wait_store(t0 + blk_per_w - 2, 0)
        wait_store(t0 + blk_per_w - 1, 1)

    return gather


def kernel(x, table):
    b, h = x.shape
    v, dim = table.shape
    xt = x.astype(jnp.int32).T
    out = _make_gather(b, h, dim)(xt, table)
    return out.transpose(2, 0, 1)


# speedup vs baseline: 1.9327x; 1.4868x over previous
import functools

import jax
import jax.numpy as jnp
from jax import lax
from jax.experimental import pallas as pl
from jax.experimental.pallas import tpu as pltpu
from jax.experimental.pallas import tpu_sc as plsc

_INFO = plsc.get_sparse_core_info()
_NW = _INFO.num_cores * _INFO.num_subcores
_L = _INFO.num_lanes
_BB = 128
_PAD = 1


@functools.lru_cache(maxsize=None)
def _make_gather(batch: int, hist: int, dim: int):
    n_blocks = (batch // _BB) * hist
    assert batch % _BB == 0 and n_blocks % (2 * _NW) == 0
    blk_per_w = n_blocks // _NW
    n_pairs = blk_per_w // 2
    bb_per_h = batch // _BB
    mesh = plsc.VectorSubcoreMesh(core_axis_name="c", subcore_axis_name="s")

    scratch = (
        [pltpu.VMEM((_BB,), jnp.int32) for _ in range(2)]
        + [pltpu.VMEM((_BB,), jnp.int32) for _ in range(2)]
        + [pltpu.VMEM((_BB, dim), jnp.float32) for _ in range(2)]
        + [pltpu.VMEM((dim, _BB + _PAD), jnp.float32) for _ in range(2)]
        + [pltpu.SemaphoreType.DMA for _ in range(6)]
    )

    @functools.partial(
        pl.kernel,
        mesh=mesh,
        compiler_params=pltpu.CompilerParams(
            use_tc_tiling_on_sc=False, needs_layout_passes=False),
        out_type=jax.ShapeDtypeStruct((hist, dim, batch), jnp.float32),
        scratch_types=scratch,
    )
    def gather(xt_hbm, table_hbm, out_hbm, *refs):
        idx_v = refs[0:2]
        line_v = refs[2:4]
        rows_v = refs[4:6]
        outb_v = refs[6:8]
        sem_i = refs[8:10]
        sem_g = refs[10:12]
        sem_s = refs[12:14]

        wid = lax.axis_index("s") * _INFO.num_cores + lax.axis_index("c")
        t0 = wid * blk_per_w
        t_last = t0 + blk_per_w - 1

        def hb(t):
            return t // bb_per_h, (t % bb_per_h) * _BB

        def start_idx(t, b):
            h, b0 = hb(t)
            return pltpu.async_copy(
                xt_hbm.at[h, pl.ds(b0, _BB)], idx_v[b], sem_i[b])

        def compute_lines(b):
            for u in range(_BB // _L):
                sl = pl.ds(u * _L, _L)
                line_v[b][sl] = idx_v[b][sl]

        def start_gather(b):
            return pltpu.async_copy(
                table_hbm.at[line_v[b]], rows_v[b], sem_g[b])

        def wait_gather(b):
            pltpu.make_async_copy(
                table_hbm.at[line_v[b]], rows_v[b], sem_g[b]).wait()

        d_ids = [lax.iota(jnp.int32, _L) + jnp.int32(q * _L)
                 for q in range(dim // _L)]

        def extract(b):
            for r in range(_BB):
                r_ids = jnp.full((_L,), r, jnp.int32)
                for q in range(dim // _L):
                    vals = rows_v[b][r, pl.ds(q * _L, _L)]
                    plsc.store_scatter(outb_v[b], [d_ids[q], r_ids], vals)

        def start_store(t, b):
            h, b0 = hb(t)
            return pltpu.async_copy(
                outb_v[b].at[:, pl.ds(0, _BB)],
                out_hbm.at[h, :, pl.ds(b0, _BB)], sem_s[b])

        def wait_store(t, b):
            h, b0 = hb(t)
            pltpu.make_async_copy(
                outb_v[b].at[:, pl.ds(0, _BB)],
                out_hbm.at[h, :, pl.ds(b0, _BB)], sem_s[b]).wait()

        start_idx(t0, 0).wait()
        compute_lines(0)
        start_gather(0)
        start_idx(t0 + 1, 1).wait()
        compute_lines(1)
        start_gather(1)
        i2 = start_idx(t0 + 2, 0)
        i3 = start_idx(t0 + 3, 1)
        wait_gather(0)
        extract(0)
        start_store(t0, 0)
        i2.wait()
        compute_lines(0)
        start_gather(0)
        wait_gather(1)
        extract(1)
        start_store(t0 + 1, 1)
        i3.wait()
        compute_lines(1)
        start_gather(1)

        def body(g, carry):
            i0 = t0 + 2 * g
            not_last = jnp.int32(g) < jnp.int32(n_pairs - 1)
            d0 = start_idx(jnp.minimum(i0 + 2, t_last), 0)
            wait_gather(0)
            wait_store(i0 - 2, 0)
            extract(0)
            start_store(i0, 0)
            d0.wait()
            compute_lines(0)

            @pl.when(not_last)
            def _():
                start_gather(0)

            d1 = start_idx(jnp.minimum(i0 + 3, t_last), 1)
            wait_gather(1)
            wait_store(i0 - 1, 1)
            extract(1)
            start_store(i0 + 1, 1)
            d1.wait()
            compute_lines(1)

            @pl.when(not_last)
            def _():
                start_gather(1)

            return carry

        lax.fori_loop(1, n_pairs, body, 0)

        wait_store(t0 + blk_per_w - 2, 0)
        wait_store(t0 + blk_per_w - 1, 1)

    return gather


def kernel(x, table):
    b, h = x.shape
    v, dim = table.shape
    xt = x.astype(jnp.int32).T
    out = _make_gather(b, h, dim)(xt, table)
    return out.transpose(2, 0, 1)


# speedup vs baseline: 2.1350x; 1.1047x over previous
import functools

import jax
import jax.numpy as jnp
from jax import lax
from jax.experimental import pallas as pl
from jax.experimental.pallas import tpu as pltpu
from jax.experimental.pallas import tpu_sc as plsc

_INFO = plsc.get_sparse_core_info()
_NW = _INFO.num_cores * _INFO.num_subcores
_L = _INFO.num_lanes
_BB = 128
_PAD = 1


@functools.lru_cache(maxsize=None)
def _make_gather(batch: int, hist: int, dim: int):
    n_blocks = (batch // _BB) * hist
    assert batch % _BB == 0 and n_blocks % (2 * _NW) == 0
    blk_per_w = n_blocks // _NW
    n_pairs = blk_per_w // 2
    bb_per_h = batch // _BB
    mesh = plsc.VectorSubcoreMesh(core_axis_name="c", subcore_axis_name="s")

    scratch = (
        [pltpu.VMEM((_BB,), jnp.int32) for _ in range(2)]
        + [pltpu.VMEM((_BB,), jnp.int32) for _ in range(2)]
        + [pltpu.VMEM((_BB, dim), jnp.float32) for _ in range(2)]
        + [pltpu.VMEM((dim // 8, 8, _BB + _PAD), jnp.float32) for _ in range(2)]
        + [pltpu.SemaphoreType.DMA for _ in range(6)]
    )

    @functools.partial(
        pl.kernel,
        mesh=mesh,
        compiler_params=pltpu.CompilerParams(
            use_tc_tiling_on_sc=False, needs_layout_passes=False),
        out_type=jax.ShapeDtypeStruct(
            (hist, dim // 8, batch // 128, 8, 128), jnp.float32),
        scratch_types=scratch,
    )
    def gather(xt_hbm, table_hbm, out_hbm, *refs):
        idx_v = refs[0:2]
        line_v = refs[2:4]
        rows_v = refs[4:6]
        outb_v = refs[6:8]
        sem_i = refs[8:10]
        sem_g = refs[10:12]
        sem_s = refs[12:14]

        wid = lax.axis_index("s") * _INFO.num_cores + lax.axis_index("c")
        t0 = wid * blk_per_w
        t_last = t0 + blk_per_w - 1

        def hb(t):
            return t // bb_per_h, (t % bb_per_h) * _BB

        def start_idx(t, b):
            h, b0 = hb(t)
            return pltpu.async_copy(
                xt_hbm.at[h, pl.ds(b0, _BB)], idx_v[b], sem_i[b])

        def compute_lines(b):
            for u in range(_BB // _L):
                sl = pl.ds(u * _L, _L)
                line_v[b][sl] = idx_v[b][sl]

        def start_gather(b):
            return pltpu.async_copy(
                table_hbm.at[line_v[b]], rows_v[b], sem_g[b])

        def wait_gather(b):
            pltpu.make_async_copy(
                table_hbm.at[line_v[b]], rows_v[b], sem_g[b]).wait()

        dt_ids = [(lax.iota(jnp.int32, _L) + jnp.int32(q * _L)) // jnp.int32(8)
                  for q in range(dim // _L)]
        ds_ids = [(lax.iota(jnp.int32, _L) + jnp.int32(q * _L)) % jnp.int32(8)
                  for q in range(dim // _L)]

        def extract(b):
            for r in range(_BB):
                r_ids = jnp.full((_L,), r, jnp.int32)
                for q in range(dim // _L):
                    vals = rows_v[b][r, pl.ds(q * _L, _L)]
                    plsc.store_scatter(
                        outb_v[b], [dt_ids[q], ds_ids[q], r_ids], vals)

        def start_store(t, b):
            h, b0 = hb(t)
            return pltpu.async_copy(
                outb_v[b].at[:, :, pl.ds(0, _BB)],
                out_hbm.at[h, :, b0 // 128, :, :], sem_s[b])

        def wait_store(t, b):
            h, b0 = hb(t)
            pltpu.make_async_copy(
                outb_v[b].at[:, :, pl.ds(0, _BB)],
                out_hbm.at[h, :, b0 // 128, :, :], sem_s[b]).wait()

        start_idx(t0, 0).wait()
        compute_lines(0)
        start_gather(0)
        start_idx(t0 + 1, 1).wait()
        compute_lines(1)
        start_gather(1)
        i2 = start_idx(t0 + 2, 0)
        i3 = start_idx(t0 + 3, 1)
        wait_gather(0)
        extract(0)
        start_store(t0, 0)
        i2.wait()
        compute_lines(0)
        start_gather(0)
        wait_gather(1)
        extract(1)
        start_store(t0 + 1, 1)
        i3.wait()
        compute_lines(1)
        start_gather(1)

        def body(g, carry):
            i0 = t0 + 2 * g
            not_last = jnp.int32(g) < jnp.int32(n_pairs - 1)
            d0 = start_idx(jnp.minimum(i0 + 2, t_last), 0)
            wait_gather(0)
            wait_store(i0 - 2, 0)
            extract(0)
            start_store(i0, 0)
            d0.wait()
            compute_lines(0)

            @pl.when(not_last)
            def _():
                start_gather(0)

            d1 = start_idx(jnp.minimum(i0 + 3, t_last), 1)
            wait_gather(1)
            wait_store(i0 - 1, 1)
            extract(1)
            start_store(i0 + 1, 1)
            d1.wait()
            compute_lines(1)

            @pl.when(not_last)
            def _():
                start_gather(1)

            return carry

        lax.fori_loop(1, n_pairs, body, 0)

        wait_store(t0 + blk_per_w - 2, 0)
        wait_store(t0 + blk_per_w - 1, 1)

    return gather


def kernel(x, table):
    b, h = x.shape
    v, dim = table.shape
    xt = x.astype(jnp.int32).T
    out5 = _make_gather(b, h, dim)(xt, table)
    return out5.transpose(2, 4, 0, 1, 3).reshape(b, h, dim)
